# baseline (speedup 1.0000x reference)
import functools

import jax
import jax.numpy as jnp
from jax import lax
from jax.experimental import pallas as pl
from jax.experimental.pallas import tpu as pltpu
from jax.experimental.pallas import tpu_sc as plsc

_L = 128
_PITCH = 136
_JW = 16


@functools.lru_cache(maxsize=None)
def _make_lookup(B, D, T, NI):
    info = plsc.get_sparse_core_info()
    NC, NS = info.num_cores, info.num_subcores
    NW = NC * NS
    n_units = B // _L
    assert n_units % NW == 0
    n_per_w = n_units // NW
    assert n_per_w % 2 == 0
    npair = n_per_w // 2
    WPJ = NI // _L
    FB = D // 8
    G = D // 16
    mesh = plsc.VectorSubcoreMesh(core_axis_name="c", subcore_axis_name="s")

    @functools.partial(
        pl.kernel,
        mesh=mesh,
        compiler_params=pltpu.CompilerParams(
            use_tc_tiling_on_sc=False, needs_layout_passes=False
        ),
        out_type=jax.ShapeDtypeStruct((T, FB, WPJ, 8, _L), jnp.float32),
        scratch_types=[
            pltpu.VMEM((NI, _JW), jnp.int32),
            pltpu.VMEM((2, _L), jnp.int32),
            pltpu.VMEM((2, _L, D), jnp.float32),
            pltpu.VMEM((2, D, _PITCH), jnp.float32),
            pltpu.SemaphoreType.DMA,
            pltpu.SemaphoreType.DMA,
            pltpu.SemaphoreType.DMA,
            pltpu.SemaphoreType.DMA,
        ],
    )
    def body(idx_hbm, table_hbm, out_hbm, idx_v, idxb, rows_v, tr_v, ga, gb, wa, wb):
        wid = lax.axis_index("s") * NC + lax.axis_index("c")
        u0 = wid * n_per_w
        j0 = lax.div(u0, WPJ)
        jb = jnp.minimum(lax.div(j0, 8) * 8, T - _JW)
        pltpu.sync_copy(idx_hbm.at[pl.ds(0, NI), pl.ds(jb, _JW)], idx_v)

        gsems = (ga, gb)
        wsems = (wa, wb)
        iota = lax.iota(jnp.int32, 16)
        row_ids = [g * 16 + iota for g in range(G)]

        def unit_coords(u):
            ug = u0 + u
            j = lax.div(ug, WPJ)
            w = lax.rem(ug, WPJ)
            return j, w

        def pack_idx(u, h):
            j, w = unit_coords(u)
            jl = jnp.broadcast_to(j - jb, (16,)).astype(jnp.int32)
            for g in range(G * 2):
                rows16 = w * _L + g * 16 + iota
                vals = plsc.load_gather(idx_v, [rows16, jl])
                idxb[h, pl.ds(g * 16, 16)] = vals

        def fire_gather(u, h):
            pltpu.async_copy(table_hbm.at[idxb.at[h]], rows_v.at[h], gsems[h])

        def wait_gather(h):
            pltpu.make_async_copy(
                table_hbm.at[pl.ds(0, _L)], rows_v.at[h], gsems[h]
            ).wait()

        def drain_wb(h):
            for fb in range(FB):
                pltpu.make_async_copy(
                    tr_v.at[h].at[pl.ds(fb * 8, 8), pl.ds(0, _L)],
                    out_hbm.at[0].at[0].at[0],
                    wsems[h],
                ).wait()

        def transpose(h):
            rows_h = rows_v.at[h]
            tr_h = tr_v.at[h]

            def irow(i4, carry):
                base = i4 * 4
                for s in range(4):
                    i = base + s
                    ib = jnp.broadcast_to(i, (16,)).astype(jnp.int32)
                    for g in range(G):
                        vals = rows_h[i, pl.ds(g * 16, 16)]
                        plsc.store_scatter(tr_h, [row_ids[g], ib], vals)
                return carry

            lax.fori_loop(0, _L // 4, irow, 0)

        def fire_wb(u, h):
            j, w = unit_coords(u)
            for fb in range(FB):
                pltpu.async_copy(
                    tr_v.at[h].at[pl.ds(fb * 8, 8), pl.ds(0, _L)],
                    out_hbm.at[j].at[fb].at[w],
                    wsems[h],
                )

        pack_idx(0, 0)
        fire_gather(0, 0)

        pack_idx(1, 1)
        fire_gather(1, 1)
        wait_gather(0)
        transpose(0)
        fire_wb(0, 0)
        pack_idx(2, 0)
        fire_gather(2, 0)
        wait_gather(1)
        transpose(1)
        fire_wb(1, 1)

        def pair(p, carry):
            u = p * 2
            pack_idx(u + 1, 1)
            fire_gather(u + 1, 1)
            wait_gather(0)
            drain_wb(0)
            transpose(0)
            fire_wb(u, 0)
            pack_idx(u + 2, 0)
            fire_gather(u + 2, 0)
            wait_gather(1)
            drain_wb(1)
            transpose(1)
            fire_wb(u + 1, 1)
            return carry

        lax.fori_loop(1, npair - 1, pair, 0)

        u = n_per_w - 2
        pack_idx(u + 1, 1)
        fire_gather(u + 1, 1)
        wait_gather(0)
        drain_wb(0)
        transpose(0)
        fire_wb(u, 0)
        wait_gather(1)
        drain_wb(1)
        transpose(1)
        fire_wb(u + 1, 1)

        drain_wb(0)
        drain_wb(1)

    return body


def kernel(input_ids, weight):
    NI, T = input_ids.shape
    D = weight.shape[1]
    B = NI * T
    idx = input_ids.astype(jnp.int32)
    out5 = _make_lookup(B, D, T, NI)(idx, weight)
    return out5.transpose(2, 4, 0, 1, 3).reshape(NI, T, D)


# speedup vs baseline: 1.0136x; 1.0136x over previous
import functools

import jax
import jax.numpy as jnp
from jax import lax
from jax.experimental import pallas as pl
from jax.experimental.pallas import tpu as pltpu
from jax.experimental.pallas import tpu_sc as plsc

_L = 128
_PITCH = 136


@functools.lru_cache(maxsize=None)
def _make_idx_prepass(T, NI):
    info = plsc.get_sparse_core_info()
    NC, NS = info.num_cores, info.num_subcores
    NW = NC * NS
    WPJ = NI // _L
    n_blocks = (T // 8) * WPJ
    assert n_blocks % NW == 0
    NB = n_blocks // NW
    mesh = plsc.VectorSubcoreMesh(core_axis_name="c", subcore_axis_name="s")

    @functools.partial(
        pl.kernel,
        mesh=mesh,
        compiler_params=pltpu.CompilerParams(use_tc_tiling_on_sc=True),
        out_type=jax.ShapeDtypeStruct((T * NI // _L, _L), jnp.int32),
        scratch_types=[
            pltpu.VMEM((NB, 8, _L), jnp.int32),
            pltpu.SemaphoreType.DMA,
            pltpu.SemaphoreType.DMA,
        ],
    )
    def body(idxt_hbm, out_hbm, buf, s_in, s_out):
        wid = lax.axis_index("s") * NC + lax.axis_index("c")
        b0 = wid * NB

        def block_coords(k):
            b = b0 + k
            return lax.div(b, WPJ), lax.rem(b, WPJ)

        for k in range(NB):
            jt, w = block_coords(k)
            pltpu.async_copy(
                idxt_hbm.at[pl.ds(jt * 8, 8), pl.ds(w * _L, _L)],
                buf.at[k],
                s_in,
            )
        for k in range(NB):
            pltpu.make_async_copy(
                idxt_hbm.at[pl.ds(0, 8), pl.ds(0, _L)], buf.at[k], s_in
            ).wait()
        for k in range(NB):
            jt, w = block_coords(k)
            for js in range(8):
                pltpu.async_copy(
                    buf.at[k].at[js],
                    out_hbm.at[(jt * 8 + js) * WPJ + w],
                    s_out,
                )
        for k in range(NB):
            for js in range(8):
                pltpu.make_async_copy(
                    buf.at[k].at[js], out_hbm.at[0], s_out
                ).wait()

    return body


@functools.lru_cache(maxsize=None)
def _make_lookup(B, D, T, NI):
    info = plsc.get_sparse_core_info()
    NC, NS = info.num_cores, info.num_subcores
    NW = NC * NS
    n_units = B // _L
    assert n_units % NW == 0
    n_per_w = n_units // NW
    assert n_per_w % 2 == 0
    npair = n_per_w // 2
    WPJ = NI // _L
    FB = D // 8
    G = D // 16
    mesh = plsc.VectorSubcoreMesh(core_axis_name="c", subcore_axis_name="s")

    @functools.partial(
        pl.kernel,
        mesh=mesh,
        compiler_params=pltpu.CompilerParams(
            use_tc_tiling_on_sc=False, needs_layout_passes=False
        ),
        out_type=jax.ShapeDtypeStruct((T, FB, WPJ, 8, _L), jnp.float32),
        scratch_types=[
            pltpu.VMEM((n_per_w, _L), jnp.int32),
            pltpu.VMEM((2, _L, D), jnp.float32),
            pltpu.VMEM((2, D, _PITCH), jnp.float32),
            pltpu.SemaphoreType.DMA,
            pltpu.SemaphoreType.DMA,
            pltpu.SemaphoreType.DMA,
            pltpu.SemaphoreType.DMA,
        ],
    )
    def body(idx_hbm, table_hbm, out_hbm, idx_v, rows_v, tr_v, ga, gb, wa, wb):
        wid = lax.axis_index("s") * NC + lax.axis_index("c")
        u0 = wid * n_per_w
        pltpu.sync_copy(idx_hbm.at[pl.ds(u0, n_per_w)], idx_v)

        gsems = (ga, gb)
        wsems = (wa, wb)
        iota = lax.iota(jnp.int32, 16)
        row_ids = [g * 16 + iota for g in range(G)]

        def unit_coords(u):
            ug = u0 + u
            j = lax.div(ug, WPJ)
            w = lax.rem(ug, WPJ)
            return j, w

        def fire_gather(u, h):
            pltpu.async_copy(table_hbm.at[idx_v.at[u]], rows_v.at[h], gsems[h])

        def wait_gather(h):
            pltpu.make_async_copy(
                table_hbm.at[pl.ds(0, _L)], rows_v.at[h], gsems[h]
            ).wait()

        def drain_wb(h):
            for fb in range(FB):
                pltpu.make_async_copy(
                    tr_v.at[h].at[pl.ds(fb * 8, 8), pl.ds(0, _L)],
                    out_hbm.at[0].at[0].at[0],
                    wsems[h],
                ).wait()

        def transpose(h):
            rows_h = rows_v.at[h]
            tr_h = tr_v.at[h]

            def irow(i4, carry):
                base = i4 * 4
                for s in range(4):
                    i = base + s
                    ib = jnp.broadcast_to(i, (16,)).astype(jnp.int32)
                    for g in range(G):
                        vals = rows_h[i, pl.ds(g * 16, 16)]
                        plsc.store_scatter(tr_h, [row_ids[g], ib], vals)
                return carry

            lax.fori_loop(0, _L // 4, irow, 0)

        def fire_wb(u, h):
            j, w = unit_coords(u)
            for fb in range(FB):
                pltpu.async_copy(
                    tr_v.at[h].at[pl.ds(fb * 8, 8), pl.ds(0, _L)],
                    out_hbm.at[j].at[fb].at[w],
                    wsems[h],
                )

        fire_gather(0, 0)

        fire_gather(1, 1)
        wait_gather(0)
        transpose(0)
        fire_wb(0, 0)
        fire_gather(2, 0)
        wait_gather(1)
        transpose(1)
        fire_wb(1, 1)

        def pair(p, carry):
            u = p * 2
            fire_gather(u + 1, 1)
            wait_gather(0)
            drain_wb(0)
            transpose(0)
            fire_wb(u, 0)
            fire_gather(u + 2, 0)
            wait_gather(1)
            drain_wb(1)
            transpose(1)
            fire_wb(u + 1, 1)
            return carry

        lax.fori_loop(1, npair - 1, pair, 0)

        u = n_per_w - 2
        fire_gather(u + 1, 1)
        wait_gather(0)
        drain_wb(0)
        transpose(0)
        fire_wb(u, 0)
        wait_gather(1)
        drain_wb(1)
        transpose(1)
        fire_wb(u + 1, 1)

        drain_wb(0)
        drain_wb(1)

    return body


def kernel(input_ids, weight):
    NI, T = input_ids.shape
    D = weight.shape[1]
    B = NI * T
    idx = _make_idx_prepass(T, NI)(input_ids.T.astype(jnp.int32))
    out5 = _make_lookup(B, D, T, NI)(idx, weight)
    return out5.transpose(2, 4, 0, 1, 3).reshape(NI, T, D)
